# baseline (speedup 1.0000x reference)
import functools

import jax
import jax.numpy as jnp
from jax import lax
from jax.experimental import pallas as pl
from jax.experimental.pallas import tpu as pltpu
from jax.experimental.pallas import tpu_sc as plsc

N = 64 * 32768
NW = 32
PER_W = N // NW
CHUNK = 16384
NCH = PER_W // CHUNK
VPC = CHUNK // 16
S = 4
ITERS = VPC // S

_mesh = plsc.VectorSubcoreMesh(core_axis_name="c", subcore_axis_name="s")


def _pairwise(v, i, v2, i2):
    p = (v2 > v) | ((v2 == v) & (i2 < i))
    return jnp.where(p, v2, v), jnp.where(p, i2, i)


def _lane_allreduce(v, i, buf_f, buf_i, lane):
    for sh in (8, 4, 2, 1):
        buf_f[...] = v
        buf_i[...] = i
        idx2 = lane ^ sh
        v2 = plsc.load_gather(buf_f, [idx2])
        i2 = plsc.load_gather(buf_i, [idx2])
        v, i = _pairwise(v, i, v2, i2)
    return v, i


@functools.partial(
    pl.kernel,
    mesh=_mesh,
    compiler_params=pltpu.CompilerParams(needs_layout_passes=False),
    out_type=[
        jax.ShapeDtypeStruct((2, 16), jnp.float32),
        jax.ShapeDtypeStruct((2, 16), jnp.int32),
    ],
    scratch_types=[
        pltpu.VMEM((CHUNK,), jnp.float32),
        pltpu.VMEM((CHUNK,), jnp.float32),
        pltpu.SemaphoreType.DMA,
        pltpu.SemaphoreType.DMA,
        pltpu.VMEM((16,), jnp.float32),
        pltpu.VMEM((16,), jnp.int32),
        pltpu.VMEM_SHARED((256,), jnp.float32),
        pltpu.VMEM_SHARED((256,), jnp.int32),
        pltpu.VMEM((256,), jnp.float32),
        pltpu.VMEM((256,), jnp.int32),
    ],
)
def _argmax_sc(A, out_val, out_idx, buf_a, buf_b, sem_a, sem_b,
               res_f, res_i, sh_val, sh_idx, mg_val, mg_idx):
    c = lax.axis_index("c")
    s = lax.axis_index("s")
    wid = c * 16 + s
    base = wid * PER_W

    bufs = (buf_a, buf_b)
    sems = (sem_a, sem_b)

    pltpu.make_async_copy(A.at[pl.ds(base, CHUNK)], bufs[0], sems[0]).start()

    neg_inf = jnp.full((16,), -jnp.inf, jnp.float32)
    zero = jnp.zeros((16,), jnp.int32)
    acc = tuple((neg_inf, zero) for _ in range(S))

    for k in range(NCH):
        buf = bufs[k % 2]
        pltpu.make_async_copy(
            A.at[pl.ds(base + k * CHUNK, CHUNK)], buf, sems[k % 2]).wait()
        if k + 1 < NCH:
            nxt = bufs[(k + 1) % 2]
            pltpu.make_async_copy(
                A.at[pl.ds(base + (k + 1) * CHUNK, CHUNK)],
                nxt, sems[(k + 1) % 2]).start()
        vec_base = base // 16 + k * VPC

        @plsc.parallel_loop(0, ITERS, 1, unroll=4, carry=acc)
        def _loop(i, carry, buf=buf, vec_base=vec_base):
            out = []
            for t in range(S):
                m, vidx = carry[t]
                j = i * S + t
                v = buf[pl.ds(j * 16, 16)]
                p = v > m
                jg = jnp.full((16,), vec_base + j, jnp.int32)
                out.append((jnp.where(p, v, m), jnp.where(p, jg, vidx)))
            return tuple(out)

        acc = _loop

    lane = lax.iota(jnp.int32, 16)
    m, fi = acc[0][0], acc[0][1] * 16 + lane
    for t in range(1, S):
        m, fi = _pairwise(m, fi, acc[t][0], acc[t][1] * 16 + lane)

    wv, wi = _lane_allreduce(m, fi, res_f, res_i, lane)

    res_f[...] = wv
    res_i[...] = wi
    pltpu.sync_copy(res_f, sh_val.at[pl.ds(s * 16, 16)])
    pltpu.sync_copy(res_i, sh_idx.at[pl.ds(s * 16, 16)])
    plsc.subcore_barrier()

    pltpu.sync_copy(sh_val, mg_val)
    pltpu.sync_copy(sh_idx, mg_idx)
    gather_idx = lane * 16
    vals = plsc.load_gather(mg_val, [gather_idx])
    idxs = plsc.load_gather(mg_idx, [gather_idx])
    cv, ci = _lane_allreduce(vals, idxs, res_f, res_i, lane)
    res_f[...] = cv
    res_i[...] = ci

    @pl.when(s == 0)
    def _():
        pltpu.sync_copy(res_f, out_val.at[c])
        pltpu.sync_copy(res_i, out_idx.at[c])


def kernel(A):
    vals, idxs = _argmax_sc(A.reshape(-1))
    v0, v1 = vals[0, 0], vals[1, 0]
    i0, i1 = idxs[0, 0], idxs[1, 0]
    take1 = (v1 > v0) | ((v1 == v0) & (i1 < i0))
    return jnp.where(take1, i1, i0).astype(jnp.int64)
